# Initial kernel scaffold; baseline (speedup 1.0000x reference)
import jax
import jax.numpy as jnp
from jax.experimental import pallas as pl


def kernel(x, A, temperature):
    raise NotImplementedError("write your pallas kernel here")


# speedup vs baseline: 12.9124x; 12.9124x over previous
import functools

import jax
import jax.numpy as jnp
from jax.experimental import pallas as pl
from jax.experimental.pallas import tpu as pltpu

B = 4
N = 1000
D = 256
K = 5
ROWS = 200


def _dgm_block(x_blk_ref, x_full_ref, t_ref, idx_ref, lp_ref):
    t = t_ref[0]
    col_iota = jax.lax.broadcasted_iota(jnp.int32, (ROWS, N), 1)

    dists = []
    lq = None
    for b in range(B):
        xb = x_blk_ref[b]
        xf = x_full_ref[b]
        g = jax.lax.dot_general(
            xb, xf, (((1,), (1,)), ((), ())),
            preferred_element_type=jnp.float32,
            precision=jax.lax.Precision.DEFAULT)
        sq_r = jnp.sum(xb * xb, axis=1, keepdims=True)
        sq_c = jnp.sum(xf * xf, axis=1)[None, :]
        d_b = sq_r + sq_c - 2.0 * g
        dists.append(d_b)
        contrib = d_b * t
        lq = contrib if lq is None else lq + contrib

    work = lq
    big = jnp.float32(jnp.inf)
    idx_cols = []
    lp_cols = [[] for _ in range(B)]
    for _ in range(K):
        m = jnp.min(work, axis=1, keepdims=True)
        sel = work == m
        idx = jnp.min(jnp.where(sel, col_iota, jnp.int32(N)), axis=1,
                      keepdims=True)
        onehot = col_iota == idx
        idx_cols.append(idx)
        for b in range(B):
            lp_cols[b].append(
                -t * jnp.sum(jnp.where(onehot, dists[b], 0.0), axis=1,
                             keepdims=True))
        work = jnp.where(onehot, big, work)

    idx_ref[...] = jnp.concatenate(idx_cols, axis=1)
    lp_ref[...] = jnp.stack(
        [jnp.concatenate(cols, axis=1) for cols in lp_cols], axis=0)


@functools.partial(jax.jit, static_argnames=())
def kernel(x, A, temperature):
    del A
    xr = x.reshape(B, N, D)
    t = jnp.exp(jnp.clip(temperature, -5.0, 5.0)).reshape(1)

    grid = (N // ROWS,)
    idx_out, lp_out = pl.pallas_call(
        _dgm_block,
        grid=grid,
        in_specs=[
            pl.BlockSpec((B, ROWS, D), lambda i: (0, i, 0)),
            pl.BlockSpec((B, N, D), lambda i: (0, 0, 0)),
            pl.BlockSpec(memory_space=pltpu.SMEM),
        ],
        out_specs=[
            pl.BlockSpec((ROWS, K), lambda i: (i, 0)),
            pl.BlockSpec((B, ROWS, K), lambda i: (0, i, 0)),
        ],
        out_shape=[
            jax.ShapeDtypeStruct((N, K), jnp.int32),
            jax.ShapeDtypeStruct((B, N, K), jnp.float32),
        ],
    )(xr, xr, t)

    flat_idx = idx_out.reshape(-1)
    src = jnp.tile(jnp.repeat(jnp.arange(N, dtype=jnp.int32), K), B)
    tgt = jnp.tile(flat_idx, B)
    offset = jnp.repeat(jnp.arange(B, dtype=jnp.int32) * N, N * K)
    edges = jnp.stack([src + offset, tgt + offset])
    return (x, edges, lp_out)
